# baseline (speedup 1.0000x reference)
import functools

import jax
import jax.numpy as jnp
from jax import lax
from jax.experimental import pallas as pl
from jax.experimental.pallas import tpu as pltpu
from jax.experimental.pallas import tpu_sc as plsc

SEL = 4096
LANES = 16
RADIX = 256


def _keys_body(n, cls_ref, keys_ref):
  m = jnp.max(cls_ref[...], axis=0)
  bits = lax.bitcast_convert_type(m, jnp.int32)
  ck = jnp.where(bits < 0, bits, ~(bits | jnp.int32(-2147483648)))
  keys_ref[:, :n] = ck
  keys_ref[:, n:] = jnp.full(
      (keys_ref.shape[0], keys_ref.shape[1] - n), -1, jnp.int32)


def _sc_body(n, npad, keys_hbm, comb_hbm, out_hbm,
             keys_a, idx_a, keys_b, idx_b, hist, idx2d, rows, sem):
  b = lax.axis_index("s") * 2 + lax.axis_index("c")
  nb = keys_hbm.shape[0]
  chunk = npad // LANES
  iota = lax.iota(jnp.int32, LANES)
  lane_base = iota * chunk
  ones = jnp.ones((LANES,), jnp.int32)

  @pl.when(b < nb)
  def _():
    pltpu.sync_copy(keys_hbm.at[b], keys_a)

    @pl.loop(0, npad // LANES)
    def _(t):
      idx_a[pl.ds(t * LANES, LANES)] = t * LANES + iota

    for p, (ks, vs, kd, vd) in enumerate((
        (keys_a, idx_a, keys_b, idx_b),
        (keys_b, idx_b, keys_a, idx_a),
        (keys_a, idx_a, keys_b, idx_b),
        (keys_b, idx_b, None, None),
    )):
      shift = 8 * p

      @pl.loop(0, RADIX)
      def _(d):
        hist[d] = jnp.zeros((LANES,), jnp.int32)

      @pl.loop(0, chunk)
      def _(t):
        k = plsc.load_gather(ks, [lane_base + t])
        d = lax.shift_right_logical(k, shift) & 0xFF
        plsc.addupdate_scatter(hist, [d, iota], ones)

      @pl.loop(0, RADIX, init_carry=jnp.int32(0))
      def _(d, run):
        v = hist[d]
        inc = plsc.cumsum(v)
        hist[d] = (inc - v) + run
        return run + jnp.sum(v)

      if kd is not None:
        @pl.loop(0, chunk)
        def _(t):
          g = lane_base + t
          k = plsc.load_gather(ks, [g])
          v = plsc.load_gather(vs, [g])
          d = lax.shift_right_logical(k, shift) & 0xFF
          pos = plsc.load_gather(hist, [d, iota])
          plsc.store_scatter(kd, [pos], k)
          plsc.store_scatter(vd, [pos], v)
          plsc.store_scatter(hist, [d, iota], pos + 1)
      else:
        @pl.loop(0, chunk)
        def _(t):
          g = lane_base + t
          k = plsc.load_gather(ks, [g])
          v = plsc.load_gather(vs, [g])
          d = lax.shift_right_logical(k, shift) & 0xFF
          pos = plsc.load_gather(hist, [d, iota])
          plsc.store_scatter(idx2d, [lax.shift_right_logical(pos, 7),
                                     pos & 127], v, mask=pos < SEL)
          plsc.store_scatter(hist, [d, iota], pos + 1)

    for q in range(SEL // 1024):
      descs = []
      for j in range(8):
        descs.append(pltpu.async_copy(
            comb_hbm.at[b].at[idx2d.at[8 * q + j]],
            rows.at[pl.ds(j * 128, 128)], sem))
      for dsc in descs:
        dsc.wait()
      pltpu.sync_copy(rows, out_hbm.at[b].at[pl.ds(q * 1024, 1024)])


def _mm(x, w):
  return lax.dot_general(x, w, (((1,), (0,)), ((), ())),
                         precision=lax.Precision.HIGHEST,
                         preferred_element_type=jnp.float32)


def _head_body(comb_ref, w1_ref, b1_ref, w2_ref, b2_ref, wb_ref, bb_ref,
               wr_ref, br_ref, bin_ref, res_ref):
  x = comb_ref[0]
  z = jnp.zeros((1, x.shape[1]), jnp.float32)
  xd = jnp.concatenate([z, x[:-1, :]], axis=0)
  xu = jnp.concatenate([x[1:, :], z], axis=0)
  w1 = w1_ref[...]
  h1 = _mm(xd, w1[0:16]) + _mm(x, w1[16:32]) + _mm(xu, w1[32:48])
  h1 = jnp.maximum(h1 + b1_ref[...], 0.0)
  z1 = jnp.zeros((1, h1.shape[1]), jnp.float32)
  h1d = jnp.concatenate([z1, h1[:-1, :]], axis=0)
  h1u = jnp.concatenate([h1[1:, :], z1], axis=0)
  w2 = w2_ref[...]
  h2 = _mm(h1d, w2[0:32]) + _mm(h1, w2[32:64]) + _mm(h1u, w2[64:96])
  h2 = jnp.maximum(h2 + b2_ref[...], 0.0)
  bin_ref[0] = _mm(h2, wb_ref[...]) + bb_ref[...]
  res_ref[0] = _mm(h2, wr_ref[...]) + br_ref[...]


def kernel(rpn_box_preds, rpn_cls_preds, batch_size, w1, g1, be1, rm1, rv1,
           w2, g2, be2, rm2, rv2, wb, bb, wr, br):
  bsz, n, _ = rpn_box_preds.shape
  npad = ((n + 127) // 128) * 128
  if npad % (LANES * 8):
    npad += LANES * 8 - npad % (LANES * 8)

  zeros1 = jnp.zeros((bsz, n, 1), jnp.float32)
  zeros5 = jnp.zeros((bsz, n, 5), jnp.float32)
  comb = jnp.concatenate([rpn_box_preds, zeros1, rpn_cls_preds, zeros5],
                         axis=-1)
  cls_t = jnp.transpose(rpn_cls_preds, (2, 0, 1))

  eps = 1e-5
  s1 = g1 * lax.rsqrt(rv1 + eps)
  wt1 = jnp.transpose(w1 * s1[:, None, None], (2, 1, 0))
  w1c = jnp.zeros((3, 16, 32), jnp.float32)
  w1c = w1c.at[:, 0:7].set(wt1[:, 0:7]).at[:, 8:11].set(wt1[:, 7:10])
  w1c = w1c.reshape(48, 32)
  b1c = be1 - rm1 * s1
  s2 = g2 * lax.rsqrt(rv2 + eps)
  w2c = jnp.transpose(w2 * s2[:, None, None], (2, 1, 0)).reshape(96, 64)
  b2c = be2 - rm2 * s2
  wb2 = wb[:, :, 0].T
  wr2 = wr[:, :, 0].T

  keys = pl.pallas_call(
      functools.partial(_keys_body, n),
      out_shape=jax.ShapeDtypeStruct((bsz, npad), jnp.int32),
  )(cls_t)

  mesh = plsc.VectorSubcoreMesh(core_axis_name="c", subcore_axis_name="s",
                                num_cores=2, num_subcores=16)
  comb_sel = pl.kernel(
      functools.partial(_sc_body, n, npad),
      out_type=jax.ShapeDtypeStruct((bsz, SEL, 16), jnp.float32),
      mesh=mesh,
      compiler_params=pltpu.CompilerParams(needs_layout_passes=False,
                                           use_tc_tiling_on_sc=False),
      scratch_types=[
          pltpu.VMEM((npad,), jnp.int32),
          pltpu.VMEM((npad,), jnp.int32),
          pltpu.VMEM((npad,), jnp.int32),
          pltpu.VMEM((npad,), jnp.int32),
          pltpu.VMEM((RADIX, LANES), jnp.int32),
          pltpu.VMEM((32, 128), jnp.int32),
          pltpu.VMEM((1024, 16), jnp.float32),
          pltpu.SemaphoreType.DMA,
      ],
  )(keys, comb)

  iou_bin, iou_res = pl.pallas_call(
      _head_body,
      grid=(bsz,),
      in_specs=[
          pl.BlockSpec((1, SEL, 16), lambda i: (i, 0, 0)),
          pl.BlockSpec((48, 32), lambda i: (0, 0)),
          pl.BlockSpec((32,), lambda i: (0,)),
          pl.BlockSpec((96, 64), lambda i: (0, 0)),
          pl.BlockSpec((64,), lambda i: (0,)),
          pl.BlockSpec((64, 5), lambda i: (0, 0)),
          pl.BlockSpec((5,), lambda i: (0,)),
          pl.BlockSpec((64, 1), lambda i: (0, 0)),
          pl.BlockSpec((1,), lambda i: (0,)),
      ],
      out_specs=[
          pl.BlockSpec((1, SEL, 5), lambda i: (i, 0, 0)),
          pl.BlockSpec((1, SEL, 1), lambda i: (i, 0, 0)),
      ],
      out_shape=[
          jax.ShapeDtypeStruct((bsz, SEL, 5), jnp.float32),
          jax.ShapeDtypeStruct((bsz, SEL, 1), jnp.float32),
      ],
  )(comb_sel, w1c, b1c, w2c, b2c, wb2, bb, wr2, br)

  box_sel = comb_sel[:, :, 0:7]
  cls_sel = comb_sel[:, :, 8:11]
  return (iou_bin, iou_res, box_sel, cls_sel)
